# Initial kernel scaffold; baseline (speedup 1.0000x reference)
import jax
import jax.numpy as jnp
from jax.experimental import pallas as pl


def kernel(x, idx):
    raise NotImplementedError("write your pallas kernel here")


# speedup vs baseline: 5.8901x; 5.8901x over previous
import jax
import jax.numpy as jnp
from jax import lax
from jax.experimental import pallas as pl
from jax.experimental.pallas import tpu as pltpu, tpu_sc as plsc

MASK = 103.0

B = 8192
D = 2048
NC = 2
NS = 16
NW = NC * NS
RPW = B // NW
CHUNK = 16
NCHUNK = RPW // CHUNK
CELEM = CHUNK * D


def _body(x_hbm, idx_hbm, out_hbm, idx_v, buf0, buf1,
          isem0, isem1, osem0, osem1):
    wid = lax.axis_index("s") * NC + lax.axis_index("c")
    base = wid * RPW

    pltpu.sync_copy(idx_hbm.at[pl.ds(base, RPW)], idx_v)

    bufs = (buf0, buf1)
    isems = (isem0, isem1)
    osems = (osem0, osem1)
    row_off = lax.iota(jnp.int32, CHUNK) * D
    vals = jnp.full((CHUNK,), MASK, dtype=jnp.float32)

    in_dma = [None, None]
    out_dma = [None, None]

    in_dma[0] = pltpu.async_copy(
        x_hbm.at[pl.ds(base * D, CELEM)], bufs[0], isems[0])

    for c in range(NCHUNK):
        b = c % 2
        nb = (c + 1) % 2
        if c + 1 < NCHUNK:
            if out_dma[nb] is not None:
                out_dma[nb].wait()
            in_dma[nb] = pltpu.async_copy(
                x_hbm.at[pl.ds((base + (c + 1) * CHUNK) * D, CELEM)],
                bufs[nb], isems[nb])
        in_dma[b].wait()
        cols = idx_v[pl.ds(c * CHUNK, CHUNK)]
        plsc.store_scatter(bufs[b], [row_off + cols], vals)
        out_dma[b] = pltpu.async_copy(
            bufs[b], out_hbm.at[pl.ds((base + c * CHUNK) * D, CELEM)],
            osems[b])

    out_dma[(NCHUNK - 2) % 2].wait()
    out_dma[(NCHUNK - 1) % 2].wait()


_sc_mask = pl.kernel(
    _body,
    out_type=jax.ShapeDtypeStruct((B * D,), jnp.float32),
    mesh=plsc.VectorSubcoreMesh(core_axis_name="c", subcore_axis_name="s"),
    compiler_params=pltpu.CompilerParams(needs_layout_passes=False),
    scratch_types=[
        pltpu.VMEM((RPW,), jnp.int32),
        pltpu.VMEM((CELEM,), jnp.float32),
        pltpu.VMEM((CELEM,), jnp.float32),
        pltpu.SemaphoreType.DMA,
        pltpu.SemaphoreType.DMA,
        pltpu.SemaphoreType.DMA,
        pltpu.SemaphoreType.DMA,
    ],
)


@jax.jit
def kernel(x, idx):
    cols = idx.reshape(B).astype(jnp.int32)
    out = _sc_mask(x.reshape(B * D), cols)
    return out.reshape(B, D)
